# baseline (speedup 1.0000x reference)
import jax
import jax.numpy as jnp
from jax.experimental import pallas as pl
from jax.experimental.pallas import tpu as pltpu

_K = 1024
_C = 256
_B = 8
_P = 1024
_BETA = 0.25
_EPS = 1e-12
_ENT_EPS = 1e-4


def _vq_body(z_ref, ew_ref, zq_ref, idx_ref, loss_ref, ent_ref,
             ewn_ref, ewthi_ref, ewtlo_ref, usage_ref):
    b = pl.program_id(0)
    nb = pl.num_programs(0)

    @pl.when(b == 0)
    def _init():
        ew = ew_ref[...]
        norm = jnp.sqrt(jnp.sum(ew * ew, axis=1, keepdims=True))
        ewn = ew / jnp.maximum(norm, _EPS)
        ewn_ref[...] = ewn
        ewt = ewn.T
        hi = ewt.astype(jnp.bfloat16)
        ewthi_ref[...] = hi
        ewtlo_ref[...] = (ewt - hi.astype(jnp.float32)).astype(jnp.bfloat16)
        usage_ref[...] = jnp.zeros_like(usage_ref)
        loss_ref[...] = jnp.zeros_like(loss_ref)

    cdims = (((1,), (0,)), ((), ()))
    z3 = z_ref[0].reshape(_C, 8, 128)
    s2 = jnp.sum(z3 * z3, axis=0)
    n = jnp.maximum(jnp.sqrt(s2), _EPS)
    zn = (z3 / n[None]).reshape(_C, _P)
    sim = jax.lax.dot_general(
        ewn_ref[...].astype(jnp.bfloat16), zn.astype(jnp.bfloat16), cdims,
        preferred_element_type=jnp.float32)
    smax = jnp.max(sim, axis=0, keepdims=True)
    kiota = jax.lax.broadcasted_iota(jnp.int32, sim.shape, 0)
    idx = jnp.min(jnp.where(sim == smax, kiota, jnp.int32(2**30)),
                  axis=0, keepdims=True)
    idx_ref[pl.ds(b, 1), :] = idx

    onehot = (kiota == idx).astype(jnp.float32)
    usage_ref[...] += jnp.sum(onehot, axis=1, keepdims=True)
    oh16 = onehot.astype(jnp.bfloat16)
    zq = (jax.lax.dot_general(ewthi_ref[...], oh16, cdims,
                              preferred_element_type=jnp.float32)
          + jax.lax.dot_general(ewtlo_ref[...], oh16, cdims,
                                preferred_element_type=jnp.float32))
    zq_ref[0] = zq.reshape(_C * 8, 128)
    diff = zq - zn
    loss_ref[...] += jnp.sum(diff * diff).reshape(1, 1)

    @pl.when(b == nb - 1)
    def _finish():
        total = jnp.float32(_B * _P)
        loss_ref[...] = (1.0 + _BETA) * (loss_ref[...] / total)
        pe = usage_ref[...] + _ENT_EPS
        probs = pe / jnp.sum(pe)
        ent_ref[...] = -jnp.sum(probs * jnp.log(probs)).reshape(1, 1)


def kernel(z, embedding_weight):
    zr = z.reshape(_B, _C * 8, 128)
    zq, idx, loss, ent = pl.pallas_call(
        _vq_body,
        grid=(_B,),
        in_specs=[
            pl.BlockSpec((1, _C * 8, 128), lambda b: (b, 0, 0)),
            pl.BlockSpec((_K, _C), lambda b: (0, 0)),
        ],
        out_specs=[
            pl.BlockSpec((1, _C * 8, 128), lambda b: (b, 0, 0)),
            pl.BlockSpec((_B, _P), lambda b: (0, 0)),
            pl.BlockSpec((1, 1), lambda b: (0, 0)),
            pl.BlockSpec((1, 1), lambda b: (0, 0)),
        ],
        out_shape=[
            jax.ShapeDtypeStruct((_B, _C * 8, 128), jnp.float32),
            jax.ShapeDtypeStruct((_B, _P), jnp.int32),
            jax.ShapeDtypeStruct((1, 1), jnp.float32),
            jax.ShapeDtypeStruct((1, 1), jnp.float32),
        ],
        scratch_shapes=[
            pltpu.VMEM((_K, _C), jnp.float32),
            pltpu.VMEM((_C, _K), jnp.bfloat16),
            pltpu.VMEM((_C, _K), jnp.bfloat16),
            pltpu.VMEM((_K, 1), jnp.float32),
        ],
    )(zr, embedding_weight)
    return (zq.reshape(_B, _C, 32, 32), loss[0, 0], ent[0, 0], idx)


# speedup vs baseline: 2.4274x; 2.4274x over previous
import jax
import jax.numpy as jnp
from jax.experimental import pallas as pl
from jax.experimental.pallas import tpu as pltpu

_K = 1024
_C = 256
_B = 8
_P = 1024
_BETA = 0.25
_EPS = 1e-12
_ENT_EPS = 1e-4


def _vq_body(z_ref, ew_ref, zq_ref, idx_ref, loss_ref, ent_ref,
             ewn_ref, ewthi_ref, ewtlo_ref, usage_ref):
    b = pl.program_id(0)
    nb = pl.num_programs(0)

    @pl.when(b == 0)
    def _init():
        ew = ew_ref[...]
        norm = jnp.sqrt(jnp.sum(ew * ew, axis=1, keepdims=True))
        ewn = ew / jnp.maximum(norm, _EPS)
        ewn_ref[...] = ewn
        ewt = ewn.T
        hi = ewt.astype(jnp.bfloat16)
        ewthi_ref[...] = hi
        ewtlo_ref[...] = (ewt - hi.astype(jnp.float32)).astype(jnp.bfloat16)
        usage_ref[...] = jnp.zeros_like(usage_ref)
        loss_ref[...] = jnp.zeros_like(loss_ref)

    cdims = (((1,), (0,)), ((), ()))
    z = z_ref[0]
    s2 = jnp.sum(z * z, axis=0, keepdims=True)
    zn = z / jnp.maximum(jnp.sqrt(s2), _EPS)
    sim = jax.lax.dot_general(
        ewn_ref[...].astype(jnp.bfloat16), zn.astype(jnp.bfloat16), cdims,
        preferred_element_type=jnp.float32)
    smax = jnp.max(sim, axis=0, keepdims=True)
    kiota = jax.lax.broadcasted_iota(jnp.int32, sim.shape, 0)
    idx = jnp.min(jnp.where(sim == smax, kiota, jnp.int32(2**30)),
                  axis=0, keepdims=True)
    idx_ref[pl.ds(b, 1), :] = idx

    onehot = (kiota == idx).astype(jnp.float32)
    usage_ref[...] += jnp.sum(onehot, axis=1, keepdims=True)
    oh16 = onehot.astype(jnp.bfloat16)
    zq = (jax.lax.dot_general(ewthi_ref[...], oh16, cdims,
                              preferred_element_type=jnp.float32)
          + jax.lax.dot_general(ewtlo_ref[...], oh16, cdims,
                                preferred_element_type=jnp.float32))
    zq_ref[0] = zq
    diff = zq - zn
    loss_ref[...] += jnp.sum(diff * diff).reshape(1, 1)

    @pl.when(b == nb - 1)
    def _finish():
        total = jnp.float32(_B * _P)
        loss_ref[...] = (1.0 + _BETA) * (loss_ref[...] / total)
        pe = usage_ref[...] + _ENT_EPS
        probs = pe / jnp.sum(pe)
        ent_ref[...] = -jnp.sum(probs * jnp.log(probs)).reshape(1, 1)


def kernel(z, embedding_weight):
    zr = z.reshape(_B, _C, _P)
    zq, idx, loss, ent = pl.pallas_call(
        _vq_body,
        grid=(_B,),
        in_specs=[
            pl.BlockSpec((1, _C, _P), lambda b: (b, 0, 0)),
            pl.BlockSpec((_K, _C), lambda b: (0, 0)),
        ],
        out_specs=[
            pl.BlockSpec((1, _C, _P), lambda b: (b, 0, 0)),
            pl.BlockSpec((_B, _P), lambda b: (0, 0)),
            pl.BlockSpec((1, 1), lambda b: (0, 0)),
            pl.BlockSpec((1, 1), lambda b: (0, 0)),
        ],
        out_shape=[
            jax.ShapeDtypeStruct((_B, _C, _P), jnp.float32),
            jax.ShapeDtypeStruct((_B, _P), jnp.int32),
            jax.ShapeDtypeStruct((1, 1), jnp.float32),
            jax.ShapeDtypeStruct((1, 1), jnp.float32),
        ],
        scratch_shapes=[
            pltpu.VMEM((_K, _C), jnp.float32),
            pltpu.VMEM((_C, _K), jnp.bfloat16),
            pltpu.VMEM((_C, _K), jnp.bfloat16),
            pltpu.VMEM((_K, 1), jnp.float32),
        ],
    )(zr, embedding_weight)
    return (zq.reshape(_B, _C, 32, 32), loss[0, 0], ent[0, 0], idx)


# speedup vs baseline: 2.5673x; 1.0577x over previous
import jax
import jax.numpy as jnp
from jax.experimental import pallas as pl
from jax.experimental.pallas import tpu as pltpu

_K = 1024
_C = 256
_B = 8
_P = 1024
_BETA = 0.25
_EPS = 1e-12
_ENT_EPS = 1e-4


def _vq_body(z_ref, ew_ref, zq_ref, idx_ref, loss_ref, ent_ref,
             ewn16_ref, ewthi_ref, ewtlo_ref, usage_ref):
    b = pl.program_id(0)
    nb = pl.num_programs(0)

    @pl.when(b == 0)
    def _init():
        ew = ew_ref[...]
        norm = jnp.sqrt(jnp.sum(ew * ew, axis=1, keepdims=True))
        ewn = ew / jnp.maximum(norm, _EPS)
        ewn16_ref[...] = ewn.astype(jnp.bfloat16)
        ewt = ewn.T
        hi = ewt.astype(jnp.bfloat16)
        ewthi_ref[...] = hi
        ewtlo_ref[...] = (ewt - hi.astype(jnp.float32)).astype(jnp.bfloat16)
        usage_ref[...] = jnp.zeros_like(usage_ref)
        loss_ref[...] = jnp.zeros_like(loss_ref)

    cdims = (((1,), (0,)), ((), ()))
    z = z_ref[0]
    s2 = jnp.sum(z * z, axis=0, keepdims=True)
    n = jnp.maximum(jnp.sqrt(s2), _EPS)
    zn16 = (z / n).astype(jnp.bfloat16)
    CK = 256
    rmax = None
    ridx = None
    for c in range(_K // CK):
        simc = jax.lax.dot_general(
            ewn16_ref[c * CK:(c + 1) * CK, :], zn16, cdims,
            preferred_element_type=jnp.float32)
        cmax = jnp.max(simc, axis=0, keepdims=True)
        kio = jax.lax.broadcasted_iota(jnp.int32, simc.shape, 0)
        cidx = jnp.min(jnp.where(simc == cmax, kio, jnp.int32(2**30)),
                       axis=0, keepdims=True) + (c * CK)
        if rmax is None:
            rmax, ridx = cmax, cidx
        else:
            better = cmax > rmax
            ridx = jnp.where(better, cidx, ridx)
            rmax = jnp.where(better, cmax, rmax)
    idx_ref[pl.ds(b, 1), :] = ridx

    kiota = jax.lax.broadcasted_iota(jnp.int32, (_K, _P), 0)
    onehot = (kiota == ridx).astype(jnp.float32)
    usage_ref[...] += jnp.sum(onehot, axis=1, keepdims=True)
    oh16 = onehot.astype(jnp.bfloat16)
    zq = (jax.lax.dot_general(ewthi_ref[...], oh16, cdims,
                              preferred_element_type=jnp.float32)
          + jax.lax.dot_general(ewtlo_ref[...], oh16, cdims,
                                preferred_element_type=jnp.float32))
    zq_ref[0] = zq
    loss_ref[...] += jnp.sum(2.0 - 2.0 * rmax).reshape(1, 1)

    @pl.when(b == nb - 1)
    def _finish():
        total = jnp.float32(_B * _P)
        loss_ref[...] = (1.0 + _BETA) * (loss_ref[...] / total)
        pe = usage_ref[...] + _ENT_EPS
        probs = pe / jnp.sum(pe)
        ent_ref[...] = -jnp.sum(probs * jnp.log(probs)).reshape(1, 1)


def kernel(z, embedding_weight):
    zr = z.reshape(_B, _C, _P)
    zq, idx, loss, ent = pl.pallas_call(
        _vq_body,
        grid=(_B,),
        in_specs=[
            pl.BlockSpec((1, _C, _P), lambda b: (b, 0, 0)),
            pl.BlockSpec((_K, _C), lambda b: (0, 0)),
        ],
        out_specs=[
            pl.BlockSpec((1, _C, _P), lambda b: (b, 0, 0)),
            pl.BlockSpec((_B, _P), lambda b: (0, 0)),
            pl.BlockSpec((1, 1), lambda b: (0, 0)),
            pl.BlockSpec((1, 1), lambda b: (0, 0)),
        ],
        out_shape=[
            jax.ShapeDtypeStruct((_B, _C, _P), jnp.float32),
            jax.ShapeDtypeStruct((_B, _P), jnp.int32),
            jax.ShapeDtypeStruct((1, 1), jnp.float32),
            jax.ShapeDtypeStruct((1, 1), jnp.float32),
        ],
        scratch_shapes=[
            pltpu.VMEM((_K, _C), jnp.bfloat16),
            pltpu.VMEM((_C, _K), jnp.bfloat16),
            pltpu.VMEM((_C, _K), jnp.bfloat16),
            pltpu.VMEM((_K, 1), jnp.float32),
        ],
    )(zr, embedding_weight)
    return (zq.reshape(_B, _C, 32, 32), loss[0, 0], ent[0, 0], idx)


# speedup vs baseline: 2.7053x; 1.0537x over previous
import jax
import jax.numpy as jnp
from jax.experimental import pallas as pl
from jax.experimental.pallas import tpu as pltpu

_K = 1024
_C = 256
_B = 8
_P = 1024
_BETA = 0.25
_EPS = 1e-12
_ENT_EPS = 1e-4
_NCORE = 2
_NS = _B // _NCORE


def _vq_body(z_ref, ew_ref, zq_ref, idx_ref, loss_ref, usage_ref,
             ewn16_ref, ewthi_ref, usage_acc):
    s = pl.program_id(1)

    @pl.when(s == 0)
    def _init():
        ew = ew_ref[...]
        norm = jnp.sqrt(jnp.sum(ew * ew, axis=1, keepdims=True))
        ewn = ew / jnp.maximum(norm, _EPS)
        ewn16_ref[...] = ewn.astype(jnp.bfloat16)
        ewthi_ref[...] = ewn.T.astype(jnp.bfloat16)
        usage_acc[...] = jnp.zeros_like(usage_acc)
        loss_ref[...] = jnp.zeros_like(loss_ref)

    cdims = (((1,), (0,)), ((), ()))
    z = z_ref[0]
    s2 = jnp.sum(z * z, axis=0, keepdims=True)
    n = jnp.maximum(jnp.sqrt(s2), _EPS)
    zn16 = (z / n).astype(jnp.bfloat16)
    CK = 256
    rmax = None
    ridx = None
    for c in range(_K // CK):
        simc = jax.lax.dot_general(
            ewn16_ref[c * CK:(c + 1) * CK, :], zn16, cdims,
            preferred_element_type=jnp.float32)
        cmax = jnp.max(simc, axis=0, keepdims=True)
        kio = jax.lax.broadcasted_iota(jnp.int32, simc.shape, 0)
        cidx = jnp.min(jnp.where(simc == cmax, kio, jnp.int32(2**30)),
                       axis=0, keepdims=True) + (c * CK)
        if rmax is None:
            rmax, ridx = cmax, cidx
        else:
            better = cmax > rmax
            ridx = jnp.where(better, cidx, ridx)
            rmax = jnp.where(better, cmax, rmax)
    idx_ref[0, pl.ds(s, 1), :] = ridx

    kiota = jax.lax.broadcasted_iota(jnp.int32, (_K, _P), 0)
    onehot = (kiota == ridx).astype(jnp.float32)
    usage_acc[...] += jnp.sum(onehot, axis=1, keepdims=True)
    zq = jax.lax.dot_general(ewthi_ref[...], onehot.astype(jnp.bfloat16),
                             cdims, preferred_element_type=jnp.float32)
    zq_ref[0] = zq
    loss_ref[...] += jnp.sum(2.0 - 2.0 * rmax).reshape(1, 1, 1)

    @pl.when(s == _NS - 1)
    def _flush():
        usage_ref[0] = usage_acc[...].reshape(1, _K)


def _fin_body(loss2_ref, usage2_ref, loss_ref, ent_ref):
    loss_ref[...] = ((1.0 + _BETA) / (_B * _P)) * jnp.sum(
        loss2_ref[...]).reshape(1, 1)
    pe = jnp.sum(usage2_ref[...], axis=0) + _ENT_EPS
    probs = pe / jnp.sum(pe)
    ent_ref[...] = -jnp.sum(probs * jnp.log(probs)).reshape(1, 1)


def kernel(z, embedding_weight):
    zr = z.reshape(_B, _C, _P)
    zq, idx, loss2, usage2 = pl.pallas_call(
        _vq_body,
        grid=(_NCORE, _NS),
        in_specs=[
            pl.BlockSpec((1, _C, _P), lambda c, s: (c * _NS + s, 0, 0)),
            pl.BlockSpec((_K, _C), lambda c, s: (0, 0)),
        ],
        out_specs=[
            pl.BlockSpec((1, _C, _P), lambda c, s: (c * _NS + s, 0, 0)),
            pl.BlockSpec((1, _NS, _P), lambda c, s: (c, 0, 0)),
            pl.BlockSpec((1, 1, 1), lambda c, s: (c, 0, 0)),
            pl.BlockSpec((1, 1, _K), lambda c, s: (c, 0, 0)),
        ],
        out_shape=[
            jax.ShapeDtypeStruct((_B, _C, _P), jnp.float32),
            jax.ShapeDtypeStruct((_NCORE, _NS, _P), jnp.int32),
            jax.ShapeDtypeStruct((_NCORE, 1, 1), jnp.float32),
            jax.ShapeDtypeStruct((_NCORE, 1, _K), jnp.float32),
        ],
        scratch_shapes=[
            pltpu.VMEM((_K, _C), jnp.bfloat16),
            pltpu.VMEM((_C, _K), jnp.bfloat16),
            pltpu.VMEM((_K, 1), jnp.float32),
        ],
        compiler_params=pltpu.CompilerParams(
            dimension_semantics=("parallel", "arbitrary")),
    )(zr, embedding_weight)
    loss, ent = pl.pallas_call(
        _fin_body,
        in_specs=[
            pl.BlockSpec((_NCORE, 1, 1), lambda: (0, 0, 0)),
            pl.BlockSpec((_NCORE, 1, _K), lambda: (0, 0, 0)),
        ],
        out_specs=[
            pl.BlockSpec((1, 1), lambda: (0, 0)),
            pl.BlockSpec((1, 1), lambda: (0, 0)),
        ],
        out_shape=[
            jax.ShapeDtypeStruct((1, 1), jnp.float32),
            jax.ShapeDtypeStruct((1, 1), jnp.float32),
        ],
    )(loss2, usage2)
    return (zq.reshape(_B, _C, 32, 32), loss[0, 0], ent[0, 0],
            idx.reshape(_B, _P))


# speedup vs baseline: 2.8206x; 1.0426x over previous
import jax
import jax.numpy as jnp
from jax.experimental import pallas as pl
from jax.experimental.pallas import tpu as pltpu

_K = 1024
_C = 256
_B = 8
_P = 1024
_BETA = 0.25
_EPS = 1e-12
_ENT_EPS = 1e-4


def _vq_body(z_ref, ew_ref, zq_ref, idx_ref, loss_ref, ent_ref,
             ewn16_ref, ewthi_ref):
    b = pl.program_id(0)

    @pl.when(b == 0)
    def _init():
        ew = ew_ref[...]
        norm = jnp.sqrt(jnp.sum(ew * ew, axis=1, keepdims=True))
        ewn = ew / jnp.maximum(norm, _EPS)
        ewn16_ref[...] = ewn.astype(jnp.bfloat16)
        ewthi_ref[...] = ewn.T.astype(jnp.bfloat16)
        loss_ref[...] = jnp.zeros_like(loss_ref)

    cdims = (((1,), (0,)), ((), ()))
    z = z_ref[0]
    s2 = jnp.sum(z * z, axis=0, keepdims=True)
    n = jnp.maximum(jnp.sqrt(s2), _EPS)
    zn16 = (z / n).astype(jnp.bfloat16)
    CK = 256
    rmax = None
    ridx = None
    for c in range(_K // CK):
        simc = jax.lax.dot_general(
            ewn16_ref[c * CK:(c + 1) * CK, :], zn16, cdims,
            preferred_element_type=jnp.float32)
        cmax = jnp.max(simc, axis=0, keepdims=True)
        kio = jax.lax.broadcasted_iota(jnp.int32, simc.shape, 0)
        cidx = jnp.min(jnp.where(simc == cmax, kio, jnp.int32(2**30)),
                       axis=0, keepdims=True) + (c * CK)
        if rmax is None:
            rmax, ridx = cmax, cidx
        else:
            better = cmax > rmax
            ridx = jnp.where(better, cidx, ridx)
            rmax = jnp.where(better, cmax, rmax)
    idx_ref[pl.ds(b, 1), :] = ridx

    kiota = jax.lax.broadcasted_iota(jnp.int32, (_K, _P), 0)
    oh16 = (kiota == ridx).astype(jnp.bfloat16)
    zq = jax.lax.dot_general(ewthi_ref[...], oh16, cdims,
                             preferred_element_type=jnp.float32)
    zq_ref[0] = zq
    loss_ref[...] += jnp.sum(2.0 - 2.0 * rmax).reshape(1, 1)

    @pl.when(b == _B - 1)
    def _finish():
        loss_ref[...] = ((1.0 + _BETA) / (_B * _P)) * loss_ref[...]
        idxall = idx_ref[...].reshape(1, _B * _P)
        kcol = jax.lax.broadcasted_iota(jnp.int32, (_K, 1), 0)
        cnt = jnp.sum((idxall == kcol).astype(jnp.float32),
                      axis=1, keepdims=True)
        pe = cnt + _ENT_EPS
        probs = pe / jnp.sum(pe)
        ent_ref[...] = -jnp.sum(probs * jnp.log(probs)).reshape(1, 1)


def kernel(z, embedding_weight):
    zr = z.reshape(_B, _C, _P)
    zq, idx, loss, ent = pl.pallas_call(
        _vq_body,
        grid=(_B,),
        in_specs=[
            pl.BlockSpec((1, _C, _P), lambda b: (b, 0, 0)),
            pl.BlockSpec((_K, _C), lambda b: (0, 0)),
        ],
        out_specs=[
            pl.BlockSpec((1, _C, _P), lambda b: (b, 0, 0)),
            pl.BlockSpec((_B, _P), lambda b: (0, 0)),
            pl.BlockSpec((1, 1), lambda b: (0, 0)),
            pl.BlockSpec((1, 1), lambda b: (0, 0)),
        ],
        out_shape=[
            jax.ShapeDtypeStruct((_B, _C, _P), jnp.float32),
            jax.ShapeDtypeStruct((_B, _P), jnp.int32),
            jax.ShapeDtypeStruct((1, 1), jnp.float32),
            jax.ShapeDtypeStruct((1, 1), jnp.float32),
        ],
        scratch_shapes=[
            pltpu.VMEM((_K, _C), jnp.bfloat16),
            pltpu.VMEM((_C, _K), jnp.bfloat16),
        ],
    )(zr, embedding_weight)
    return (zq.reshape(_B, _C, 32, 32), loss[0, 0], ent[0, 0], idx)
